# baseline (speedup 1.0000x reference)
import functools

import jax
import jax.numpy as jnp
from jax import lax
from jax.experimental import pallas as pl
from jax.experimental.pallas import tpu as pltpu
from jax.experimental.pallas import tpu_sc as plsc

NC = 2
NS = 16
L = 16
NW = NC * NS
B_C = 128
N_C0 = 148
N_C1 = 12


def _sc_body(d_feat, iu_hbm, ip_hbm, xu_hbm, xp_hbm, out_hbm,
             idxu_v, idxp_v, u0, p0, u1, p1, out_v, sem0, sem1):
    cid = lax.axis_index("c")
    sid = lax.axis_index("s")
    n_sub = d_feat // (2 * L)
    iota = lax.iota(jnp.int32, L)
    bufs = ((u0, p0, sem0), (u1, p1, sem1))

    def worker(base_chunk, nck):
        n_w = nck * B_C
        base_e = base_chunk * B_C
        pltpu.sync_copy(iu_hbm.at[pl.ds(base_e, n_w)],
                        idxu_v.at[pl.ds(0, n_w)])
        pltpu.sync_copy(ip_hbm.at[pl.ds(base_e, n_w)],
                        idxp_v.at[pl.ds(0, n_w)])

        def fire(c, u_b, p_b, sem_b):
            off = c * B_C
            pltpu.async_copy(
                xu_hbm.at[idxu_v.at[pl.ds(off, B_C)]], u_b, sem_b)
            pltpu.async_copy(
                xp_hbm.at[idxp_v.at[pl.ds(off, B_C)]], p_b, sem_b)

        def drain(u_b, p_b, sem_b):
            pltpu.make_async_copy(
                xu_hbm.at[pl.ds(0, B_C)], u_b, sem_b).wait()
            pltpu.make_async_copy(
                xp_hbm.at[pl.ds(0, B_C)], p_b, sem_b).wait()

        def compute(c, u_b, p_b):
            def grp_body(g, carry):
                rb = g * L
                s = jnp.zeros((L,), jnp.float32)
                for i in range(L):
                    r = rb + i
                    acc = jnp.zeros((L,), jnp.float32)
                    for j in range(n_sub):
                        u2 = u_b[r, pl.ds(j * 2 * L, 2 * L)]
                        p2 = p_b[r, pl.ds(j * 2 * L, 2 * L)]
                        ua, ub2 = plsc.unpack(
                            u2, format=plsc.PackFormat.INTERLEAVED)
                        pa, pb2 = plsc.unpack(
                            p2, format=plsc.PackFormat.INTERLEAVED)
                        acc = acc + ua * pa + ub2 * pb2
                    d = lax.reduce_sum_p.bind(acc, axes=(0,))
                    s = jnp.where(iota == i, d, s)
                out_v[pl.ds(c * B_C + rb, L)] = s
                return carry

            lax.fori_loop(0, B_C // L, grp_body, 0)

        fire(0, *bufs[0])
        fire(1, *bufs[1])

        def pair_body(t, carry):
            for b in range(2):
                c = 2 * t + b
                u_b, p_b, sem_b = bufs[b]
                drain(u_b, p_b, sem_b)
                compute(c, u_b, p_b)

                @pl.when(c + 2 < nck)
                def _():
                    fire(c + 2, u_b, p_b, sem_b)
            return carry

        lax.fori_loop(0, nck // 2, pair_body, 0)
        pltpu.sync_copy(out_v.at[pl.ds(0, n_w)],
                        out_hbm.at[pl.ds(base_e, n_w)])

    @pl.when(cid == 0)
    def _():
        worker(sid * N_C0, N_C0)

    @pl.when(cid == 1)
    def _():
        worker(NS * N_C0 + sid * N_C1, N_C1)


@functools.partial(jax.jit, static_argnames=("d_feat",))
def _sc_gather_dot(iu, ip, x_user, x_product, d_feat):
    n_pad = iu.shape[0]
    n_max = max(N_C0, N_C1) * B_C
    mesh = plsc.VectorSubcoreMesh(core_axis_name="c", subcore_axis_name="s")
    return pl.kernel(
        functools.partial(_sc_body, d_feat),
        out_type=jax.ShapeDtypeStruct((n_pad,), jnp.float32),
        mesh=mesh,
        compiler_params=pltpu.CompilerParams(
            needs_layout_passes=False, use_tc_tiling_on_sc=False),
        scratch_types=[
            pltpu.VMEM((n_max,), jnp.int32),
            pltpu.VMEM((n_max,), jnp.int32),
            pltpu.VMEM((B_C, d_feat), jnp.bfloat16),
            pltpu.VMEM((B_C, d_feat), jnp.bfloat16),
            pltpu.VMEM((B_C, d_feat), jnp.bfloat16),
            pltpu.VMEM((B_C, d_feat), jnp.bfloat16),
            pltpu.VMEM((n_max,), jnp.float32),
            pltpu.SemaphoreType.DMA,
            pltpu.SemaphoreType.DMA,
        ],
    )(iu, ip, x_user, x_product)


def kernel(x_user, x_product, edge_label_index):
    x_user = x_user.astype(jnp.bfloat16)
    x_product = x_product.astype(jnp.bfloat16)
    n_edges = edge_label_index.shape[1]
    d_feat = x_user.shape[1]
    n_pad = NS * (N_C0 + N_C1) * B_C
    idx = edge_label_index.astype(jnp.int32)
    iu = jnp.pad(idx[0], (0, n_pad - n_edges))
    ip = jnp.pad(idx[1], (0, n_pad - n_edges))
    out = _sc_gather_dot(iu, ip, x_user, x_product, d_feat)
    return out[:n_edges]


# speedup vs baseline: 1.0614x; 1.0614x over previous
import functools

import jax
import jax.numpy as jnp
from jax import lax
from jax.experimental import pallas as pl
from jax.experimental.pallas import tpu as pltpu
from jax.experimental.pallas import tpu_sc as plsc

NC = 2
NS = 16
L = 16
NW = NC * NS
B_C = 128
N_C0 = 128
N_C1 = 32


def _sc_body(d_feat, iu_hbm, ip_hbm, xu_hbm, xp_hbm, out_hbm,
             idxu_v, idxp_v, u0, p0, u1, p1, out_v, sem0, sem1):
    cid = lax.axis_index("c")
    sid = lax.axis_index("s")
    n_sub = d_feat // (2 * L)
    iota = lax.iota(jnp.int32, L)
    bufs = ((u0, p0, sem0), (u1, p1, sem1))

    def worker(base_chunk, nck):
        n_w = nck * B_C
        base_e = base_chunk * B_C
        pltpu.sync_copy(iu_hbm.at[pl.ds(base_e, n_w)],
                        idxu_v.at[pl.ds(0, n_w)])
        pltpu.sync_copy(ip_hbm.at[pl.ds(base_e, n_w)],
                        idxp_v.at[pl.ds(0, n_w)])

        def fire(c, u_b, p_b, sem_b):
            off = c * B_C
            pltpu.async_copy(
                xu_hbm.at[idxu_v.at[pl.ds(off, B_C)]], u_b, sem_b)
            pltpu.async_copy(
                xp_hbm.at[idxp_v.at[pl.ds(off, B_C)]], p_b, sem_b)

        def drain(u_b, p_b, sem_b):
            pltpu.make_async_copy(
                xu_hbm.at[pl.ds(0, B_C)], u_b, sem_b).wait()
            pltpu.make_async_copy(
                xp_hbm.at[pl.ds(0, B_C)], p_b, sem_b).wait()

        def compute(c, u_b, p_b):
            def grp_body(g, carry):
                rb = g * L
                s = jnp.zeros((L,), jnp.float32)
                for i in range(L):
                    r = rb + i
                    acc = jnp.zeros((L,), jnp.float32)
                    for j in range(n_sub):
                        u2 = u_b[r, pl.ds(j * 2 * L, 2 * L)]
                        p2 = p_b[r, pl.ds(j * 2 * L, 2 * L)]
                        ua, ub2 = plsc.unpack(
                            u2, format=plsc.PackFormat.INTERLEAVED)
                        pa, pb2 = plsc.unpack(
                            p2, format=plsc.PackFormat.INTERLEAVED)
                        acc = acc + ua * pa + ub2 * pb2
                    d = lax.reduce_sum_p.bind(acc, axes=(0,))
                    s = jnp.where(iota == i, d, s)
                out_v[pl.ds(c * B_C + rb, L)] = s
                return carry

            lax.fori_loop(0, B_C // L, grp_body, 0)

        fire(0, *bufs[0])
        fire(1, *bufs[1])

        def pair_body(t, carry):
            for b in range(2):
                c = 2 * t + b
                u_b, p_b, sem_b = bufs[b]
                drain(u_b, p_b, sem_b)
                compute(c, u_b, p_b)

                @pl.when(c + 2 < nck)
                def _():
                    fire(c + 2, u_b, p_b, sem_b)
            return carry

        lax.fori_loop(0, nck // 2, pair_body, 0)
        pltpu.sync_copy(out_v.at[pl.ds(0, n_w)],
                        out_hbm.at[pl.ds(base_e, n_w)])

    @pl.when(cid == 0)
    def _():
        worker(sid * N_C0, N_C0)

    @pl.when(cid == 1)
    def _():
        worker(NS * N_C0 + sid * N_C1, N_C1)


@functools.partial(jax.jit, static_argnames=("d_feat",))
def _sc_gather_dot(iu, ip, x_user, x_product, d_feat):
    n_pad = iu.shape[0]
    n_max = max(N_C0, N_C1) * B_C
    mesh = plsc.VectorSubcoreMesh(core_axis_name="c", subcore_axis_name="s")
    return pl.kernel(
        functools.partial(_sc_body, d_feat),
        out_type=jax.ShapeDtypeStruct((n_pad,), jnp.float32),
        mesh=mesh,
        compiler_params=pltpu.CompilerParams(
            needs_layout_passes=False, use_tc_tiling_on_sc=False),
        scratch_types=[
            pltpu.VMEM((n_max,), jnp.int32),
            pltpu.VMEM((n_max,), jnp.int32),
            pltpu.VMEM((B_C, d_feat), jnp.bfloat16),
            pltpu.VMEM((B_C, d_feat), jnp.bfloat16),
            pltpu.VMEM((B_C, d_feat), jnp.bfloat16),
            pltpu.VMEM((B_C, d_feat), jnp.bfloat16),
            pltpu.VMEM((n_max,), jnp.float32),
            pltpu.SemaphoreType.DMA,
            pltpu.SemaphoreType.DMA,
        ],
    )(iu, ip, x_user, x_product)


def kernel(x_user, x_product, edge_label_index):
    x_user = x_user.astype(jnp.bfloat16)
    x_product = x_product.astype(jnp.bfloat16)
    n_edges = edge_label_index.shape[1]
    d_feat = x_user.shape[1]
    n_pad = NS * (N_C0 + N_C1) * B_C
    idx = edge_label_index.astype(jnp.int32)
    iu = jnp.pad(idx[0], (0, n_pad - n_edges))
    ip = jnp.pad(idx[1], (0, n_pad - n_edges))
    out = _sc_gather_dot(iu, ip, x_user, x_product, d_feat)
    return out[:n_edges]


# speedup vs baseline: 1.0746x; 1.0124x over previous
import functools

import jax
import jax.numpy as jnp
from jax import lax
from jax.experimental import pallas as pl
from jax.experimental.pallas import tpu as pltpu
from jax.experimental.pallas import tpu_sc as plsc

NC = 2
NS = 16
L = 16
NW = NC * NS
B_C = 128
N_C0 = 140
N_C1 = 20


def _sc_body(d_feat, iu_hbm, ip_hbm, xu_hbm, xp_hbm, out_hbm,
             idxu_v, idxp_v, u0, p0, u1, p1, out_v, sem0, sem1):
    cid = lax.axis_index("c")
    sid = lax.axis_index("s")
    n_sub = d_feat // (2 * L)
    iota = lax.iota(jnp.int32, L)
    bufs = ((u0, p0, sem0), (u1, p1, sem1))

    def worker(base_chunk, nck):
        n_w = nck * B_C
        base_e = base_chunk * B_C
        pltpu.sync_copy(iu_hbm.at[pl.ds(base_e, n_w)],
                        idxu_v.at[pl.ds(0, n_w)])
        pltpu.sync_copy(ip_hbm.at[pl.ds(base_e, n_w)],
                        idxp_v.at[pl.ds(0, n_w)])

        def fire(c, u_b, p_b, sem_b):
            off = c * B_C
            pltpu.async_copy(
                xu_hbm.at[idxu_v.at[pl.ds(off, B_C)]], u_b, sem_b)
            pltpu.async_copy(
                xp_hbm.at[idxp_v.at[pl.ds(off, B_C)]], p_b, sem_b)

        def drain(u_b, p_b, sem_b):
            pltpu.make_async_copy(
                xu_hbm.at[pl.ds(0, B_C)], u_b, sem_b).wait()
            pltpu.make_async_copy(
                xp_hbm.at[pl.ds(0, B_C)], p_b, sem_b).wait()

        def compute(c, u_b, p_b):
            def grp_body(g, carry):
                rb = g * L
                s = jnp.zeros((L,), jnp.float32)
                for i in range(L):
                    r = rb + i
                    acc = jnp.zeros((L,), jnp.float32)
                    for j in range(n_sub):
                        u2 = u_b[r, pl.ds(j * 2 * L, 2 * L)]
                        p2 = p_b[r, pl.ds(j * 2 * L, 2 * L)]
                        ua, ub2 = plsc.unpack(
                            u2, format=plsc.PackFormat.INTERLEAVED)
                        pa, pb2 = plsc.unpack(
                            p2, format=plsc.PackFormat.INTERLEAVED)
                        acc = acc + ua * pa + ub2 * pb2
                    d = lax.reduce_sum_p.bind(acc, axes=(0,))
                    s = jnp.where(iota == i, d, s)
                out_v[pl.ds(c * B_C + rb, L)] = s
                return carry

            lax.fori_loop(0, B_C // L, grp_body, 0)

        fire(0, *bufs[0])
        fire(1, *bufs[1])

        def pair_body(t, carry):
            for b in range(2):
                c = 2 * t + b
                u_b, p_b, sem_b = bufs[b]
                drain(u_b, p_b, sem_b)
                compute(c, u_b, p_b)

                @pl.when(c + 2 < nck)
                def _():
                    fire(c + 2, u_b, p_b, sem_b)
            return carry

        lax.fori_loop(0, nck // 2, pair_body, 0)
        pltpu.sync_copy(out_v.at[pl.ds(0, n_w)],
                        out_hbm.at[pl.ds(base_e, n_w)])

    @pl.when(cid == 0)
    def _():
        worker(sid * N_C0, N_C0)

    @pl.when(cid == 1)
    def _():
        worker(NS * N_C0 + sid * N_C1, N_C1)


@functools.partial(jax.jit, static_argnames=("d_feat",))
def _sc_gather_dot(iu, ip, x_user, x_product, d_feat):
    n_pad = iu.shape[0]
    n_max = max(N_C0, N_C1) * B_C
    mesh = plsc.VectorSubcoreMesh(core_axis_name="c", subcore_axis_name="s")
    return pl.kernel(
        functools.partial(_sc_body, d_feat),
        out_type=jax.ShapeDtypeStruct((n_pad,), jnp.float32),
        mesh=mesh,
        compiler_params=pltpu.CompilerParams(
            needs_layout_passes=False, use_tc_tiling_on_sc=False),
        scratch_types=[
            pltpu.VMEM((n_max,), jnp.int32),
            pltpu.VMEM((n_max,), jnp.int32),
            pltpu.VMEM((B_C, d_feat), jnp.bfloat16),
            pltpu.VMEM((B_C, d_feat), jnp.bfloat16),
            pltpu.VMEM((B_C, d_feat), jnp.bfloat16),
            pltpu.VMEM((B_C, d_feat), jnp.bfloat16),
            pltpu.VMEM((n_max,), jnp.float32),
            pltpu.SemaphoreType.DMA,
            pltpu.SemaphoreType.DMA,
        ],
    )(iu, ip, x_user, x_product)


def kernel(x_user, x_product, edge_label_index):
    x_user = x_user.astype(jnp.bfloat16)
    x_product = x_product.astype(jnp.bfloat16)
    n_edges = edge_label_index.shape[1]
    d_feat = x_user.shape[1]
    n_pad = NS * (N_C0 + N_C1) * B_C
    idx = edge_label_index.astype(jnp.int32)
    iu = jnp.pad(idx[0], (0, n_pad - n_edges))
    ip = jnp.pad(idx[1], (0, n_pad - n_edges))
    out = _sc_gather_dot(iu, ip, x_user, x_product, d_feat)
    return out[:n_edges]


# speedup vs baseline: 1.0908x; 1.0151x over previous
import functools

import jax
import jax.numpy as jnp
from jax import lax
from jax.experimental import pallas as pl
from jax.experimental.pallas import tpu as pltpu
from jax.experimental.pallas import tpu_sc as plsc

NC = 2
NS = 16
L = 16
NW = NC * NS
B_C = 128
N_C0 = 138
N_C1 = 22


def _sc_body(d_feat, iu_hbm, ip_hbm, xu_hbm, xp_hbm, out_hbm,
             idxu_v, idxp_v, u0, p0, u1, p1, out_v, sem0, sem1):
    cid = lax.axis_index("c")
    sid = lax.axis_index("s")
    n_sub = d_feat // (2 * L)
    iota = lax.iota(jnp.int32, L)
    bufs = ((u0, p0, sem0), (u1, p1, sem1))

    def worker(base_chunk, nck):
        n_w = nck * B_C
        base_e = base_chunk * B_C
        pltpu.sync_copy(iu_hbm.at[pl.ds(base_e, n_w)],
                        idxu_v.at[pl.ds(0, n_w)])
        pltpu.sync_copy(ip_hbm.at[pl.ds(base_e, n_w)],
                        idxp_v.at[pl.ds(0, n_w)])

        def fire(c, u_b, p_b, sem_b):
            off = c * B_C
            pltpu.async_copy(
                xu_hbm.at[idxu_v.at[pl.ds(off, B_C)]], u_b, sem_b)
            pltpu.async_copy(
                xp_hbm.at[idxp_v.at[pl.ds(off, B_C)]], p_b, sem_b)

        def drain(u_b, p_b, sem_b):
            pltpu.make_async_copy(
                xu_hbm.at[pl.ds(0, B_C)], u_b, sem_b).wait()
            pltpu.make_async_copy(
                xp_hbm.at[pl.ds(0, B_C)], p_b, sem_b).wait()

        def compute(c, u_b, p_b):
            def grp_body(g, carry):
                rb = g * L
                s = jnp.zeros((L,), jnp.float32)
                for i in range(L):
                    r = rb + i
                    acc = jnp.zeros((L,), jnp.float32)
                    for j in range(n_sub):
                        u2 = u_b[r, pl.ds(j * 2 * L, 2 * L)]
                        p2 = p_b[r, pl.ds(j * 2 * L, 2 * L)]
                        ua, ub2 = plsc.unpack(
                            u2, format=plsc.PackFormat.INTERLEAVED)
                        pa, pb2 = plsc.unpack(
                            p2, format=plsc.PackFormat.INTERLEAVED)
                        acc = acc + ua * pa + ub2 * pb2
                    d = lax.reduce_sum_p.bind(acc, axes=(0,))
                    s = jnp.where(iota == i, d, s)
                out_v[pl.ds(c * B_C + rb, L)] = s
                return carry

            lax.fori_loop(0, B_C // L, grp_body, 0)

        fire(0, *bufs[0])
        fire(1, *bufs[1])

        def pair_body(t, carry):
            for b in range(2):
                c = 2 * t + b
                u_b, p_b, sem_b = bufs[b]
                drain(u_b, p_b, sem_b)
                compute(c, u_b, p_b)

                @pl.when(c + 2 < nck)
                def _():
                    fire(c + 2, u_b, p_b, sem_b)
            return carry

        lax.fori_loop(0, nck // 2, pair_body, 0)
        pltpu.sync_copy(out_v.at[pl.ds(0, n_w)],
                        out_hbm.at[pl.ds(base_e, n_w)])

    @pl.when(cid == 0)
    def _():
        worker(sid * N_C0, N_C0)

    @pl.when(cid == 1)
    def _():
        worker(NS * N_C0 + sid * N_C1, N_C1)


@functools.partial(jax.jit, static_argnames=("d_feat",))
def _sc_gather_dot(iu, ip, x_user, x_product, d_feat):
    n_pad = iu.shape[0]
    n_max = max(N_C0, N_C1) * B_C
    mesh = plsc.VectorSubcoreMesh(core_axis_name="c", subcore_axis_name="s")
    return pl.kernel(
        functools.partial(_sc_body, d_feat),
        out_type=jax.ShapeDtypeStruct((n_pad,), jnp.float32),
        mesh=mesh,
        compiler_params=pltpu.CompilerParams(
            needs_layout_passes=False, use_tc_tiling_on_sc=False),
        scratch_types=[
            pltpu.VMEM((n_max,), jnp.int32),
            pltpu.VMEM((n_max,), jnp.int32),
            pltpu.VMEM((B_C, d_feat), jnp.bfloat16),
            pltpu.VMEM((B_C, d_feat), jnp.bfloat16),
            pltpu.VMEM((B_C, d_feat), jnp.bfloat16),
            pltpu.VMEM((B_C, d_feat), jnp.bfloat16),
            pltpu.VMEM((n_max,), jnp.float32),
            pltpu.SemaphoreType.DMA,
            pltpu.SemaphoreType.DMA,
        ],
    )(iu, ip, x_user, x_product)


def kernel(x_user, x_product, edge_label_index):
    x_user = x_user.astype(jnp.bfloat16)
    x_product = x_product.astype(jnp.bfloat16)
    n_edges = edge_label_index.shape[1]
    d_feat = x_user.shape[1]
    n_pad = NS * (N_C0 + N_C1) * B_C
    idx = edge_label_index.astype(jnp.int32)
    iu = jnp.pad(idx[0], (0, n_pad - n_edges))
    ip = jnp.pad(idx[1], (0, n_pad - n_edges))
    out = _sc_gather_dot(iu, ip, x_user, x_product, d_feat)
    return out[:n_edges]
